# Initial kernel scaffold; baseline (speedup 1.0000x reference)
import jax
import jax.numpy as jnp
from jax.experimental import pallas as pl


def kernel(img, index_flat_inv):
    raise NotImplementedError("write your pallas kernel here")


# speedup vs baseline: 4.3082x; 4.3082x over previous
import functools

import jax
import jax.numpy as jnp
from jax import lax
from jax.experimental import pallas as pl
from jax.experimental.pallas import tpu as pltpu
from jax.experimental.pallas import tpu_sc as plsc

B, T, C = 32, 1024, 768
CHUNK = 64
NCH = T // CHUNK
NW = 32


def _loc_ed_body(img_hbm, idx_hbm, out_hbm, idx_v, buf0, buf1, sem0, sem1):
    cid = lax.axis_index("c")
    sid = lax.axis_index("s")
    b = sid * 2 + cid

    pltpu.sync_copy(idx_hbm, idx_v)

    bufs = (buf0, buf1)
    sems = (sem0, sem1)
    scat = [None, None]
    for j in range(NCH):
        k = j % 2
        if scat[k] is not None:
            scat[k].wait()
        pltpu.sync_copy(img_hbm.at[b].at[pl.ds(j * CHUNK, CHUNK)], bufs[k])
        scat[k] = pltpu.async_copy(bufs[k], out_hbm.at[b].at[idx_v.at[j]],
                                   sems[k])
    scat[0].wait()
    scat[1].wait()


@functools.partial(
    pl.kernel,
    out_type=jax.ShapeDtypeStruct((B, T, C), jnp.float32),
    mesh=plsc.VectorSubcoreMesh(core_axis_name="c", subcore_axis_name="s"),
    scratch_types=[
        pltpu.VMEM((NCH, CHUNK), jnp.int32),
        pltpu.VMEM((CHUNK, C), jnp.float32),
        pltpu.VMEM((CHUNK, C), jnp.float32),
        pltpu.SemaphoreType.DMA,
        pltpu.SemaphoreType.DMA,
    ],
)
def _loc_ed_sc(img_hbm, idx_hbm, out_hbm, idx_v, buf0, buf1, sem0, sem1):
    _loc_ed_body(img_hbm, idx_hbm, out_hbm, idx_v, buf0, buf1, sem0, sem1)


def kernel(img, index_flat_inv):
    idx32 = index_flat_inv.astype(jnp.int32).reshape(NCH, CHUNK)
    return _loc_ed_sc(img, idx32)
